# baseline (speedup 1.0000x reference)
import jax
import jax.numpy as jnp
from jax import lax
from jax.experimental import pallas as pl
from jax.experimental.pallas import tpu as pltpu
from jax.experimental.pallas import tpu_sc as plsc

N = 10000
D = 512
E = 50000
NUM_ETYPES = 3

N_SC = 2
N_TILES = 16
BATCH = 125
NBATCH = 25
STRIPE = 624
TAIL = N - N_TILES * STRIPE
HW = 256


def _sc_body(x0, x1, src_r, dst_r, zrows, ones,
             ga, gb, gc, gc2,
             acc, sidx, didx, rows0, rows1, gsem, ssem0, ssem1):
  c = lax.axis_index("c")
  s = lax.axis_index("s")
  rs = s * STRIPE

  xs = [x0, x1]
  gs = [ga, gb]

  def zero():
    pltpu.sync_copy(zrows, acc.at[pl.ds(rs, STRIPE)])

    @pl.when(s == N_TILES - 1)
    def _():
      pltpu.sync_copy(zrows.at[pl.ds(0, TAIL)], acc.at[pl.ds(N - TAIL, TAIL)])

  def copyout(e, gh):
    pltpu.sync_copy(acc.at[pl.ds(rs, STRIPE)], gh.at[e, pl.ds(rs, STRIPE)])

    @pl.when(s == N_TILES - 1)
    def _():
      pltpu.sync_copy(acc.at[pl.ds(N - TAIL, TAIL)],
                      gh.at[e, pl.ds(N - TAIL, TAIL)])

  def scatter(xh):
    def gd(j, rb):
      return pltpu.make_async_copy(xh.at[sidx.at[j]], rb, gsem)

    def sd(j, rb, sem):
      return pltpu.make_async_copy(rb, acc.at[didx.at[j]], sem)

    def pair(j, first):
      if not first:
        sd(j, rows0, ssem0).wait()
      gd(j, rows0).start()
      gd(j, rows0).wait()
      sd(j, rows0, ssem0).start(add=True)
      if not first:
        sd(j + 1, rows1, ssem1).wait()
      gd(j + 1, rows1).start()
      gd(j + 1, rows1).wait()
      sd(j + 1, rows1, ssem1).start(add=True)

    pair(0, True)

    @pl.loop(1, (NBATCH - 1) // 2)
    def _steady(i):
      pair(2 * i, False)

    j = NBATCH - 1
    sd(j, rows0, ssem0).wait()
    gd(j, rows0).start()
    gd(j, rows0).wait()
    sd(j, rows0, ssem0).start(add=True)
    sd(j, rows1, ssem1).wait()
    sd(j, rows0, ssem0).wait()

  def scatter_ones(lo, hi):
    @pl.loop(lo, hi)
    def _fire(j):
      pltpu.make_async_copy(rows0, acc.at[didx.at[j]], ssem0).start(add=True)

    @pl.loop(lo, hi)
    def _drain(j):
      pltpu.make_async_copy(rows0, acc.at[didx.at[j]], ssem0).wait()

  for e in range(NUM_ETYPES):
    pltpu.sync_copy(src_r.at[e, s], sidx)
    pltpu.sync_copy(dst_r.at[e, s], didx)

    zero()
    plsc.subcore_barrier()
    for cv in range(N_SC):
      @pl.when(c == cv)
      def _(cv=cv):
        scatter(xs[cv])
    plsc.subcore_barrier()
    for cv in range(N_SC):
      @pl.when(c == cv)
      def _(cv=cv, e=e):
        copyout(e, gs[cv])

    zero()
    pltpu.sync_copy(ones, rows0)
    plsc.subcore_barrier()
    half = (NBATCH + 1) // 2
    for cv, (lo, hi) in enumerate(((0, half), (half, NBATCH))):
      @pl.when(c == cv)
      def _(lo=lo, hi=hi):
        scatter_ones(lo, hi)
    plsc.subcore_barrier()
    for cv, gh in enumerate((gc, gc2)):
      @pl.when(c == cv)
      def _(gh=gh, e=e):
        copyout(e, gh)


_sc_scatter = pl.kernel(
    _sc_body,
    out_type=[
        jax.ShapeDtypeStruct((NUM_ETYPES, N, 2, 128), jnp.bfloat16),
        jax.ShapeDtypeStruct((NUM_ETYPES, N, 2, 128), jnp.bfloat16),
        jax.ShapeDtypeStruct((NUM_ETYPES, N, 2, 128), jnp.bfloat16),
        jax.ShapeDtypeStruct((NUM_ETYPES, N, 2, 128), jnp.bfloat16),
    ],
    mesh=plsc.VectorSubcoreMesh(core_axis_name="c", subcore_axis_name="s"),
    compiler_params=pltpu.CompilerParams(use_tc_tiling_on_sc=False),
    scratch_types=[
        pltpu.VMEM_SHARED((N, 2, 128), jnp.bfloat16),
        pltpu.VMEM((NBATCH, BATCH), jnp.int32),
        pltpu.VMEM((NBATCH, BATCH), jnp.int32),
        pltpu.VMEM((BATCH, 2, 128), jnp.bfloat16),
        pltpu.VMEM((BATCH, 2, 128), jnp.bfloat16),
        pltpu.SemaphoreType.DMA,
        pltpu.SemaphoreType.DMA,
        pltpu.SemaphoreType.DMA,
    ],
)


_BN = 400


def _mm_body(ga, gb, gc, gc2, w, b, o):
  gs = (ga, gb)
  out = jnp.zeros_like(o)
  for e in range(NUM_ETYPES):
    acc = jnp.zeros((_BN, D), jnp.float32)
    for k in range(2):
      for si in range(2):
        acc += jnp.dot(gs[k][e][:, si, :],
                       w[e, k * HW + si * 128:k * HW + (si + 1) * 128, :],
                       preferred_element_type=jnp.float32)
    cnt = (gc[e][:, 0, 0:1] + gc2[e][:, 0, 0:1]).astype(jnp.float32)
    inv = 1.0 / jnp.maximum(cnt, 1.0)
    mask = (cnt > 0.0).astype(jnp.float32)
    out += acc * inv + mask * b[e]
  o[...] = out


_mm = pl.pallas_call(
    _mm_body,
    grid=(N // _BN,),
    in_specs=[
        pl.BlockSpec((NUM_ETYPES, _BN, 2, 128), lambda r: (0, r, 0, 0)),
        pl.BlockSpec((NUM_ETYPES, _BN, 2, 128), lambda r: (0, r, 0, 0)),
        pl.BlockSpec((NUM_ETYPES, _BN, 2, 128), lambda r: (0, r, 0, 0)),
        pl.BlockSpec((NUM_ETYPES, _BN, 2, 128), lambda r: (0, r, 0, 0)),
        pl.BlockSpec((NUM_ETYPES, D, D), lambda r: (0, 0, 0)),
        pl.BlockSpec((NUM_ETYPES, 1, D), lambda r: (0, 0, 0)),
    ],
    out_specs=pl.BlockSpec((_BN, D), lambda r: (r, 0)),
    out_shape=jax.ShapeDtypeStruct((N, D), jnp.float32),
    compiler_params=pltpu.CompilerParams(
        dimension_semantics=("parallel",)),
)


@jax.jit
def kernel(x, edge_index_e0, edge_index_e1, edge_index_e2,
           W_e0, b_e0, W_e1, b_e1, W_e2, b_e2):
  xbf = x.astype(jnp.bfloat16)
  xchunks = [xbf[:, :HW].reshape(N, 2, 128), xbf[:, HW:].reshape(N, 2, 128)]

  eis = jnp.stack([edge_index_e0, edge_index_e1, edge_index_e2])
  eis = eis.reshape(NUM_ETYPES, 2, N_TILES, NBATCH, BATCH)
  src_r = eis[:, 0]
  dst_r = eis[:, 1]

  zrows = jnp.zeros((STRIPE, 2, 128), jnp.bfloat16)
  ones = jnp.ones((BATCH, 2, 128), jnp.bfloat16)

  ga, gb, gc, gc2 = _sc_scatter(*xchunks, src_r, dst_r, zrows, ones)

  w = jnp.stack([W_e0, W_e1, W_e2]).astype(jnp.bfloat16)
  b = jnp.stack([b_e0, b_e1, b_e2]).reshape(NUM_ETYPES, 1, D)
  return _mm(ga, gb, gc, gc2, w, b)


# speedup vs baseline: 1.5189x; 1.5189x over previous
import jax
import jax.numpy as jnp
from jax import lax
from jax.experimental import pallas as pl
from jax.experimental.pallas import tpu as pltpu
from jax.experimental.pallas import tpu_sc as plsc

N = 10000
D = 512
E = 50000
NUM_ETYPES = 3

N_SC = 2
N_TILES = 16
BATCH = 125
NBATCH = 25
EPT = NBATCH * BATCH
STRIPE = 624
TAIL = N - N_TILES * STRIPE


def _sc_body(x0, x1, x2, x3, src_r, dst_r, z128, ones,
             g0, g1, g2, g3, gc, gc2,
             acc, sidx, didx, rows0, rows1, gsem, ssem0, ssem1):
  c = lax.axis_index("c")
  s = lax.axis_index("s")
  rs = s * STRIPE

  xs = [x0, x1, x2, x3]
  gs = [g0, g1, g2, g3]

  def zero(a, zbuf):
    pltpu.sync_copy(zbuf, a.at[pl.ds(rs, STRIPE)])

    @pl.when(s == N_TILES - 1)
    def _():
      pltpu.sync_copy(zbuf.at[pl.ds(0, TAIL)], a.at[pl.ds(N - TAIL, TAIL)])

  def scatter(xh, a):
    def gd(j, rb):
      return pltpu.make_async_copy(xh.at[sidx.at[j]], rb, gsem)

    def sd(j, rb, sem):
      return pltpu.make_async_copy(rb, a.at[didx.at[j]], sem)

    def pair(j, first):
      if not first:
        sd(j, rows0, ssem0).wait()
      gd(j, rows0).start()
      gd(j, rows0).wait()
      sd(j, rows0, ssem0).start(add=True)
      if not first:
        sd(j + 1, rows1, ssem1).wait()
      gd(j + 1, rows1).start()
      gd(j + 1, rows1).wait()
      sd(j + 1, rows1, ssem1).start(add=True)

    pair(0, True)

    @pl.loop(1, (NBATCH - 1) // 2)
    def _steady(i):
      pair(2 * i, False)

    j = NBATCH - 1
    sd(j, rows0, ssem0).wait()
    gd(j, rows0).start()
    gd(j, rows0).wait()
    sd(j, rows0, ssem0).start(add=True)
    sd(j, rows1, ssem1).wait()
    sd(j, rows0, ssem0).wait()

  def scatter_ones(a, lo, hi):
    @pl.loop(lo, hi)
    def _fire(j):
      pltpu.make_async_copy(rows0, a.at[didx.at[j]], ssem0).start(add=True)

    @pl.loop(lo, hi)
    def _drain(j):
      pltpu.make_async_copy(rows0, a.at[didx.at[j]], ssem0).wait()

  def copyout(gh, a):
    pltpu.sync_copy(a.at[pl.ds(rs, STRIPE)], gh.at[pl.ds(rs, STRIPE)])

    @pl.when(s == N_TILES - 1)
    def _():
      pltpu.sync_copy(a.at[pl.ds(N - TAIL, TAIL)],
                      gh.at[pl.ds(N - TAIL, TAIL)])

  pltpu.sync_copy(src_r.at[s], sidx)
  pltpu.sync_copy(dst_r.at[s], didx)
  for slot in range(3):
    if slot < 2:
      zero(acc, z128)
      plsc.subcore_barrier()
      for cv in range(N_SC):
        @pl.when(c == cv)
        def _(cv=cv, slot=slot):
          scatter(xs[2 * cv + slot], acc)
      plsc.subcore_barrier()
      for cv in range(N_SC):
        @pl.when(c == cv)
        def _(cv=cv, slot=slot):
          copyout(gs[2 * cv + slot], acc)
    else:
      zero(acc, z128)
      pltpu.sync_copy(ones, rows0)
      plsc.subcore_barrier()
      half = (NBATCH + 1) // 2
      for cv, (lo, hi) in enumerate(((0, half), (half, NBATCH))):
        @pl.when(c == cv)
        def _(lo=lo, hi=hi):
          scatter_ones(acc, lo, hi)
      plsc.subcore_barrier()
      for cv, gh in enumerate((gc, gc2)):
        @pl.when(c == cv)
        def _(gh=gh):
          copyout(gh, acc)


_sc_scatter = pl.kernel(
    _sc_body,
    out_type=[jax.ShapeDtypeStruct((N, 128), jnp.float32)] * 6,
    mesh=plsc.VectorSubcoreMesh(core_axis_name="c", subcore_axis_name="s"),
    scratch_types=[
        pltpu.VMEM_SHARED((N, 128), jnp.float32),
        pltpu.VMEM((NBATCH, BATCH), jnp.int32),
        pltpu.VMEM((NBATCH, BATCH), jnp.int32),
        pltpu.VMEM((BATCH, 128), jnp.float32),
        pltpu.VMEM((BATCH, 128), jnp.float32),
        pltpu.SemaphoreType.DMA,
        pltpu.SemaphoreType.DMA,
        pltpu.SemaphoreType.DMA,
    ],
)


_BN = 400


def _mm_body(g0, g1, g2, g3, gc, gc2, w, b, o):
  gs = (g0, g1, g2, g3)
  acc = jnp.zeros_like(o)
  for k in range(4):
    acc += jnp.dot(gs[k][...], w[k * 128:(k + 1) * 128, :],
                   preferred_element_type=jnp.float32)
  cnt = gc[:, 0:1] + gc2[:, 0:1]
  inv = 1.0 / jnp.maximum(cnt, 1.0)
  mask = (cnt > 0.0).astype(jnp.float32)
  o[...] = acc * inv + mask * b[...]


_mm1 = pl.pallas_call(
    _mm_body,
    grid=(N // _BN,),
    in_specs=[pl.BlockSpec((_BN, 128), lambda r: (r, 0))] * 6 + [
        pl.BlockSpec((D, D), lambda r: (0, 0)),
        pl.BlockSpec((1, D), lambda r: (0, 0)),
    ],
    out_specs=pl.BlockSpec((_BN, D), lambda r: (r, 0)),
    out_shape=jax.ShapeDtypeStruct((N, D), jnp.float32),
    compiler_params=pltpu.CompilerParams(
        dimension_semantics=("parallel",)),
)


@jax.jit
def kernel(x, edge_index_e0, edge_index_e1, edge_index_e2,
           W_e0, b_e0, W_e1, b_e1, W_e2, b_e2):
  xchunks = [x[:, k * 128:(k + 1) * 128] for k in range(4)]

  z128 = jnp.zeros((STRIPE, 128), jnp.float32)
  ones = jnp.ones((BATCH, 128), jnp.float32)

  ws = (W_e0, W_e1, W_e2)
  bs = (b_e0.reshape(1, D), b_e1.reshape(1, D), b_e2.reshape(1, D))
  h = None
  for e, ei in enumerate((edge_index_e0, edge_index_e1, edge_index_e2)):
    er = ei.reshape(2, N_TILES, NBATCH, BATCH)
    gsx = _sc_scatter(*xchunks, er[0], er[1], z128, ones)
    he = _mm1(*gsx, ws[e], bs[e])
    h = he if h is None else h + he
  return h


# speedup vs baseline: 1.6612x; 1.0937x over previous
import jax
import jax.numpy as jnp
from jax import lax
from jax.experimental import pallas as pl
from jax.experimental.pallas import tpu as pltpu
from jax.experimental.pallas import tpu_sc as plsc

N = 10000
D = 512
E = 50000
NUM_ETYPES = 3

N_SC = 2
N_TILES = 16
BATCH = 125
NBATCH = 25
EPT = NBATCH * BATCH
STRIPE = 624
ZR = 48
TAIL = N - N_TILES * STRIPE


def _sc_body(x0, x1, x2, x3, src_r, dst_r, z128, ones,
             g0, g1, g2, g3, gc, gc2,
             acc, sidx, didx, rows0, rows1, zv, gsem, ssem0, ssem1):
  c = lax.axis_index("c")
  s = lax.axis_index("s")
  rs = s * STRIPE

  xs = [x0, x1, x2, x3]
  gs = [g0, g1, g2, g3]

  pltpu.sync_copy(z128, zv)

  def zero(a, zbuf):
    for k in range(STRIPE // ZR):
      pltpu.sync_copy(zv, a.at[pl.ds(rs + k * ZR, ZR)])

    @pl.when(s == N_TILES - 1)
    def _():
      pltpu.sync_copy(zv.at[pl.ds(0, TAIL)], a.at[pl.ds(N - TAIL, TAIL)])

  def scatter(xh, a):
    def gd(j, rb):
      return pltpu.make_async_copy(xh.at[sidx.at[j]], rb, gsem)

    def sd(j, rb, sem):
      return pltpu.make_async_copy(rb, a.at[didx.at[j]], sem)

    def pair(j, first):
      if not first:
        sd(j, rows0, ssem0).wait()
      gd(j, rows0).start()
      gd(j, rows0).wait()
      sd(j, rows0, ssem0).start(add=True)
      if not first:
        sd(j + 1, rows1, ssem1).wait()
      gd(j + 1, rows1).start()
      gd(j + 1, rows1).wait()
      sd(j + 1, rows1, ssem1).start(add=True)

    pair(0, True)

    @pl.loop(1, (NBATCH - 1) // 2)
    def _steady(i):
      pair(2 * i, False)

    j = NBATCH - 1
    sd(j, rows0, ssem0).wait()
    gd(j, rows0).start()
    gd(j, rows0).wait()
    sd(j, rows0, ssem0).start(add=True)
    sd(j, rows1, ssem1).wait()
    sd(j, rows0, ssem0).wait()

  def scatter_ones(a, lo, hi):
    @pl.loop(lo, hi)
    def _fire(j):
      pltpu.make_async_copy(rows0, a.at[didx.at[j]], ssem0).start(add=True)

    @pl.loop(lo, hi)
    def _drain(j):
      pltpu.make_async_copy(rows0, a.at[didx.at[j]], ssem0).wait()

  def copyout(e, gh, a):
    pltpu.sync_copy(a.at[pl.ds(rs, STRIPE)], gh.at[e, pl.ds(rs, STRIPE)])

    @pl.when(s == N_TILES - 1)
    def _():
      pltpu.sync_copy(a.at[pl.ds(N - TAIL, TAIL)],
                      gh.at[e, pl.ds(N - TAIL, TAIL)])

  for e in range(NUM_ETYPES):
    pltpu.sync_copy(src_r.at[e, s], sidx)
    pltpu.sync_copy(dst_r.at[e, s], didx)
    for slot in range(3):
      if slot < 2:
        zero(acc, z128)
        plsc.subcore_barrier()
        for cv in range(N_SC):
          @pl.when(c == cv)
          def _(cv=cv, slot=slot):
            scatter(xs[2 * cv + slot], acc)
        plsc.subcore_barrier()
        for cv in range(N_SC):
          @pl.when(c == cv)
          def _(cv=cv, slot=slot, e=e):
            copyout(e, gs[2 * cv + slot], acc)
      else:
        zero(acc, z128)
        pltpu.sync_copy(ones, rows0)
        plsc.subcore_barrier()
        half = (NBATCH + 1) // 2
        for cv, (lo, hi) in enumerate(((0, half), (half, NBATCH))):
          @pl.when(c == cv)
          def _(lo=lo, hi=hi):
            scatter_ones(acc, lo, hi)
        plsc.subcore_barrier()
        for cv, gh in enumerate((gc, gc2)):
          @pl.when(c == cv)
          def _(gh=gh, e=e):
            copyout(e, gh, acc)


_sc_scatter = pl.kernel(
    _sc_body,
    out_type=[
        jax.ShapeDtypeStruct((NUM_ETYPES, N, 128), jnp.float32),
        jax.ShapeDtypeStruct((NUM_ETYPES, N, 128), jnp.float32),
        jax.ShapeDtypeStruct((NUM_ETYPES, N, 128), jnp.float32),
        jax.ShapeDtypeStruct((NUM_ETYPES, N, 128), jnp.float32),
        jax.ShapeDtypeStruct((NUM_ETYPES, N, 128), jnp.float32),
        jax.ShapeDtypeStruct((NUM_ETYPES, N, 128), jnp.float32),
    ],
    mesh=plsc.VectorSubcoreMesh(core_axis_name="c", subcore_axis_name="s"),
    scratch_types=[
        pltpu.VMEM_SHARED((N, 128), jnp.float32),
        pltpu.VMEM((NBATCH, BATCH), jnp.int32),
        pltpu.VMEM((NBATCH, BATCH), jnp.int32),
        pltpu.VMEM((BATCH, 128), jnp.float32),
        pltpu.VMEM((BATCH, 128), jnp.float32),
        pltpu.VMEM((48, 128), jnp.float32),
        pltpu.SemaphoreType.DMA,
        pltpu.SemaphoreType.DMA,
        pltpu.SemaphoreType.DMA,
    ],
)


_BN = 400


def _mm_body(g0, g1, g2, g3, gc, gc2, w, b, o):
  gs = (g0, g1, g2, g3)
  out = jnp.zeros_like(o)
  for e in range(NUM_ETYPES):
    acc = jnp.zeros_like(o)
    for k in range(4):
      acc += jnp.dot(gs[k][e], w[e, k * 128:(k + 1) * 128, :],
                     preferred_element_type=jnp.float32)
    cnt = gc[e][:, 0:1] + gc2[e][:, 0:1]
    inv = 1.0 / jnp.maximum(cnt, 1.0)
    mask = (cnt > 0.0).astype(jnp.float32)
    out += acc * inv + mask * b[e]
  o[...] = out


_mm = pl.pallas_call(
    _mm_body,
    grid=(N // _BN,),
    in_specs=[
        pl.BlockSpec((NUM_ETYPES, _BN, 128), lambda r: (0, r, 0)),
        pl.BlockSpec((NUM_ETYPES, _BN, 128), lambda r: (0, r, 0)),
        pl.BlockSpec((NUM_ETYPES, _BN, 128), lambda r: (0, r, 0)),
        pl.BlockSpec((NUM_ETYPES, _BN, 128), lambda r: (0, r, 0)),
        pl.BlockSpec((NUM_ETYPES, _BN, 128), lambda r: (0, r, 0)),
        pl.BlockSpec((NUM_ETYPES, _BN, 128), lambda r: (0, r, 0)),
        pl.BlockSpec((NUM_ETYPES, D, D), lambda r: (0, 0, 0)),
        pl.BlockSpec((NUM_ETYPES, 1, D), lambda r: (0, 0, 0)),
    ],
    out_specs=pl.BlockSpec((_BN, D), lambda r: (r, 0)),
    out_shape=jax.ShapeDtypeStruct((N, D), jnp.float32),
    compiler_params=pltpu.CompilerParams(
        dimension_semantics=("parallel",)),
)


@jax.jit
def kernel(x, edge_index_e0, edge_index_e1, edge_index_e2,
           W_e0, b_e0, W_e1, b_e1, W_e2, b_e2):
  xchunks = [x[:, k * 128:(k + 1) * 128] for k in range(4)]

  eis = jnp.stack([edge_index_e0, edge_index_e1, edge_index_e2])
  eis = eis.reshape(NUM_ETYPES, 2, N_TILES, NBATCH, BATCH)
  src_r = eis[:, 0]
  dst_r = eis[:, 1]

  z128 = jnp.zeros((48, 128), jnp.float32)
  ones = jnp.ones((BATCH, 128), jnp.float32)

  g0, g1, g2, g3, gc, gc2 = _sc_scatter(*xchunks, src_r, dst_r, z128, ones)

  w = jnp.stack([W_e0, W_e1, W_e2])
  b = jnp.stack([b_e0, b_e1, b_e2]).reshape(NUM_ETYPES, 1, D)
  return _mm(g0, g1, g2, g3, gc, gc2, w, b)


# speedup vs baseline: 1.6812x; 1.0121x over previous
import jax
import jax.numpy as jnp
from jax import lax
from jax.experimental import pallas as pl
from jax.experimental.pallas import tpu as pltpu
from jax.experimental.pallas import tpu_sc as plsc

N = 10000
D = 512
E = 50000
NUM_ETYPES = 3

N_SC = 2
N_TILES = 16
BATCH = 125
NBATCH = 25
EPT = NBATCH * BATCH
STRIPE = 624
ZR = 48
TAIL = N - N_TILES * STRIPE


def _sc_body(x0, x1, x2, x3, src_r, dst_r, z128, ones,
             g0, g1, g2, g3, gc, gc2,
             acc, sidx, didx, rows0, rows1, zv, gsem, ssem0, ssem1):
  c = lax.axis_index("c")
  s = lax.axis_index("s")
  rs = s * STRIPE

  xs = [x0, x1, x2, x3]
  gs = [g0, g1, g2, g3]

  pltpu.sync_copy(z128, zv)

  def zero(a, zbuf):
    for k in range(STRIPE // ZR):
      pltpu.make_async_copy(zv, a.at[pl.ds(rs + k * ZR, ZR)], ssem0).start()

    @pl.when(s == N_TILES - 1)
    def _():
      pltpu.sync_copy(zv.at[pl.ds(0, TAIL)], a.at[pl.ds(N - TAIL, TAIL)])

    for k in range(STRIPE // ZR):
      pltpu.make_async_copy(zv, a.at[pl.ds(rs + k * ZR, ZR)], ssem0).wait()

  def scatter(xh, a):
    def gd(j, rb):
      return pltpu.make_async_copy(xh.at[sidx.at[j]], rb, gsem)

    def sd(j, rb, sem):
      return pltpu.make_async_copy(rb, a.at[didx.at[j]], sem)

    def pair(j, first):
      if not first:
        sd(j, rows0, ssem0).wait()
      gd(j, rows0).start()
      gd(j, rows0).wait()
      sd(j, rows0, ssem0).start(add=True)
      if not first:
        sd(j + 1, rows1, ssem1).wait()
      gd(j + 1, rows1).start()
      gd(j + 1, rows1).wait()
      sd(j + 1, rows1, ssem1).start(add=True)

    pair(0, True)

    @pl.loop(1, (NBATCH - 1) // 2)
    def _steady(i):
      pair(2 * i, False)

    j = NBATCH - 1
    sd(j, rows0, ssem0).wait()
    gd(j, rows0).start()
    gd(j, rows0).wait()
    sd(j, rows0, ssem0).start(add=True)
    sd(j, rows1, ssem1).wait()
    sd(j, rows0, ssem0).wait()

  def scatter_ones(a, lo, hi):
    @pl.loop(lo, hi)
    def _fire(j):
      pltpu.make_async_copy(rows0, a.at[didx.at[j]], ssem0).start(add=True)

    @pl.loop(lo, hi)
    def _drain(j):
      pltpu.make_async_copy(rows0, a.at[didx.at[j]], ssem0).wait()

  def copyout(e, gh, a):
    pltpu.sync_copy(a.at[pl.ds(rs, STRIPE)], gh.at[e, pl.ds(rs, STRIPE)])

    @pl.when(s == N_TILES - 1)
    def _():
      pltpu.sync_copy(a.at[pl.ds(N - TAIL, TAIL)],
                      gh.at[e, pl.ds(N - TAIL, TAIL)])

  for e in range(NUM_ETYPES):
    pltpu.sync_copy(src_r.at[e, s], sidx)
    pltpu.sync_copy(dst_r.at[e, s], didx)
    for slot in range(3):
      if slot < 2:
        zero(acc, z128)
        plsc.subcore_barrier()
        for cv in range(N_SC):
          @pl.when(c == cv)
          def _(cv=cv, slot=slot):
            scatter(xs[2 * cv + slot], acc)
        plsc.subcore_barrier()
        for cv in range(N_SC):
          @pl.when(c == cv)
          def _(cv=cv, slot=slot, e=e):
            copyout(e, gs[2 * cv + slot], acc)
      else:
        zero(acc, z128)
        pltpu.sync_copy(ones, rows0)
        plsc.subcore_barrier()
        half = (NBATCH + 1) // 2
        for cv, (lo, hi) in enumerate(((0, half), (half, NBATCH))):
          @pl.when(c == cv)
          def _(lo=lo, hi=hi):
            scatter_ones(acc, lo, hi)
        plsc.subcore_barrier()
        for cv, gh in enumerate((gc, gc2)):
          @pl.when(c == cv)
          def _(gh=gh, e=e):
            copyout(e, gh, acc)


_sc_scatter = pl.kernel(
    _sc_body,
    out_type=[
        jax.ShapeDtypeStruct((NUM_ETYPES, N, 128), jnp.float32),
        jax.ShapeDtypeStruct((NUM_ETYPES, N, 128), jnp.float32),
        jax.ShapeDtypeStruct((NUM_ETYPES, N, 128), jnp.float32),
        jax.ShapeDtypeStruct((NUM_ETYPES, N, 128), jnp.float32),
        jax.ShapeDtypeStruct((NUM_ETYPES, N, 128), jnp.float32),
        jax.ShapeDtypeStruct((NUM_ETYPES, N, 128), jnp.float32),
    ],
    mesh=plsc.VectorSubcoreMesh(core_axis_name="c", subcore_axis_name="s"),
    scratch_types=[
        pltpu.VMEM_SHARED((N, 128), jnp.float32),
        pltpu.VMEM((NBATCH, BATCH), jnp.int32),
        pltpu.VMEM((NBATCH, BATCH), jnp.int32),
        pltpu.VMEM((BATCH, 128), jnp.float32),
        pltpu.VMEM((BATCH, 128), jnp.float32),
        pltpu.VMEM((48, 128), jnp.float32),
        pltpu.SemaphoreType.DMA,
        pltpu.SemaphoreType.DMA,
        pltpu.SemaphoreType.DMA,
    ],
)


_BN = 400


def _mm_body(g0, g1, g2, g3, gc, gc2, w, b, o):
  gs = (g0, g1, g2, g3)
  out = jnp.zeros_like(o)
  for e in range(NUM_ETYPES):
    acc = jnp.zeros_like(o)
    for k in range(4):
      acc += jnp.dot(gs[k][e], w[e, k * 128:(k + 1) * 128, :],
                     preferred_element_type=jnp.float32)
    cnt = gc[e][:, 0:1] + gc2[e][:, 0:1]
    inv = 1.0 / jnp.maximum(cnt, 1.0)
    mask = (cnt > 0.0).astype(jnp.float32)
    out += acc * inv + mask * b[e]
  o[...] = out


_mm = pl.pallas_call(
    _mm_body,
    grid=(N // _BN,),
    in_specs=[
        pl.BlockSpec((NUM_ETYPES, _BN, 128), lambda r: (0, r, 0)),
        pl.BlockSpec((NUM_ETYPES, _BN, 128), lambda r: (0, r, 0)),
        pl.BlockSpec((NUM_ETYPES, _BN, 128), lambda r: (0, r, 0)),
        pl.BlockSpec((NUM_ETYPES, _BN, 128), lambda r: (0, r, 0)),
        pl.BlockSpec((NUM_ETYPES, _BN, 128), lambda r: (0, r, 0)),
        pl.BlockSpec((NUM_ETYPES, _BN, 128), lambda r: (0, r, 0)),
        pl.BlockSpec((NUM_ETYPES, D, D), lambda r: (0, 0, 0)),
        pl.BlockSpec((NUM_ETYPES, 1, D), lambda r: (0, 0, 0)),
    ],
    out_specs=pl.BlockSpec((_BN, D), lambda r: (r, 0)),
    out_shape=jax.ShapeDtypeStruct((N, D), jnp.float32),
    compiler_params=pltpu.CompilerParams(
        dimension_semantics=("parallel",)),
)


@jax.jit
def kernel(x, edge_index_e0, edge_index_e1, edge_index_e2,
           W_e0, b_e0, W_e1, b_e1, W_e2, b_e2):
  xchunks = [x[:, k * 128:(k + 1) * 128] for k in range(4)]

  eis = jnp.stack([edge_index_e0, edge_index_e1, edge_index_e2])
  eis = eis.reshape(NUM_ETYPES, 2, N_TILES, NBATCH, BATCH)
  src_r = eis[:, 0]
  dst_r = eis[:, 1]

  z128 = jnp.zeros((48, 128), jnp.float32)
  ones = jnp.ones((BATCH, 128), jnp.float32)

  g0, g1, g2, g3, gc, gc2 = _sc_scatter(*xchunks, src_r, dst_r, z128, ones)

  w = jnp.stack([W_e0, W_e1, W_e2])
  b = jnp.stack([b_e0, b_e1, b_e2]).reshape(NUM_ETYPES, 1, D)
  return _mm(g0, g1, g2, g3, gc, gc2, w, b)
